# baseline (speedup 1.0000x reference)
import functools

import jax
import jax.numpy as jnp
from jax import lax
from jax.experimental import pallas as pl
from jax.experimental.pallas import tpu as pltpu
from jax.experimental.pallas import tpu_sc as plsc

N_NODES = 10000
D_FEAT = 128
D_HID = 128
N_CLASSES = 32

NC = 2
NS = 16
NW = NC * NS
L = 16

NPAD = 10112
SLAB = NPAD // NS
CHUNK = 128
TOTAL_CH = 2560
E_PAD = TOTAL_CH * CHUNK

CA128, CB128 = 126, 34
CA32, CB32 = 106, 54
CAD, CBD = 102, 58

_mesh = plsc.VectorSubcoreMesh(core_axis_name="c", subcore_axis_name="s")
_sc_params = pltpu.CompilerParams(needs_layout_passes=False,
                                  use_tc_tiling_on_sc=False)


def _deg_body(idx_hbm, out_hbm, idx_v, deg_v):
    c = lax.axis_index("c")
    s = lax.axis_index("s")
    wid = s * NC + c
    zero16 = jnp.zeros((L,), jnp.float32)
    one16 = jnp.ones((L,), jnp.float32)

    def zero_body(i, carry):
        deg_v[pl.ds(i * L, L)] = zero16
        return carry

    lax.fori_loop(0, NPAD // L, zero_body, 0)

    def count(base_row, ncw):
        pltpu.sync_copy(idx_hbm.at[1, pl.ds(base_row, ncw)],
                        idx_v.at[pl.ds(0, ncw)])

        def count_chunk(i, carry):
            for j in range(CHUNK // L):
                idx = idx_v[i, pl.ds(j * L, L)]
                plsc.addupdate_scatter(deg_v, [idx], one16)
            return carry

        lax.fori_loop(0, ncw, count_chunk, 0)

    @pl.when(c == 0)
    def _():
        count(s * CAD, CAD)

    @pl.when(c == 1)
    def _():
        count(NS * CAD + s * CBD, CBD)

    pltpu.sync_copy(deg_v, out_hbm.at[wid])


_deg_kernel = functools.partial(
    pl.kernel,
    out_type=jax.ShapeDtypeStruct((NW, NPAD), jnp.float32),
    mesh=_mesh,
    compiler_params=_sc_params,
    scratch_types=[
        pltpu.VMEM((CAD, CHUNK), jnp.int32),
        pltpu.VMEM((NPAD,), jnp.float32),
    ],
)(_deg_body)


def _make_scatter(D, ca, cb):
    def body(table_hbm, idx_hbm, out_hbm, sidx_v, d0, d1, b0, b1, y_sh,
             sem_i, sem_g, sem_s):
        c = lax.axis_index("c")
        s = lax.axis_index("s")
        dbufs = (d0, d1)
        bufs = (b0, b1)
        zero16 = jnp.zeros((L,), jnp.float32)
        izero16 = jnp.zeros((L,), jnp.int32)

        def zero_row(i, carry):
            for j in range(D // L):
                b1[i, pl.ds(j * L, L)] = zero16
            return carry

        lax.fori_loop(0, CHUNK, zero_row, 0)
        for j in range(CHUNK // L):
            d1[pl.ds(j * L, L)] = izero16
        for j in range(SLAB // CHUNK):
            pltpu.sync_copy(b1, y_sh.at[pl.ds(s * SLAB + j * CHUNK, CHUNK)])
        rem = SLAB - (SLAB // CHUNK) * CHUNK
        if rem:
            pltpu.sync_copy(
                b1.at[pl.ds(0, rem)],
                y_sh.at[pl.ds(s * SLAB + SLAB - rem, rem)])
        plsc.subcore_barrier()

        def emit(base_row, ncw):
            pltpu.sync_copy(idx_hbm.at[0, pl.ds(base_row, ncw)],
                            sidx_v.at[pl.ds(0, ncw)])
            pltpu.async_copy(b1, y_sh.at[d1], sem_s, add=True)
            pltpu.async_copy(idx_hbm.at[1, base_row], d0, sem_i)
            pltpu.async_copy(table_hbm.at[sidx_v.at[0]], b0, sem_g)

            def group(j, carry):
                for p in range(2):
                    i = 2 * j + p
                    pltpu.make_async_copy(
                        idx_hbm.at[1, 0], dbufs[p], sem_i).wait()
                    pltpu.make_async_copy(
                        table_hbm.at[sidx_v.at[0]], bufs[p], sem_g).wait()
                    pltpu.make_async_copy(
                        bufs[p], y_sh.at[dbufs[p]], sem_s).wait()
                    pltpu.async_copy(
                        bufs[p], y_sh.at[dbufs[p]], sem_s, add=True)

                    if p == 0:
                        pltpu.async_copy(
                            idx_hbm.at[1, base_row + i + 1], dbufs[1], sem_i)
                        pltpu.async_copy(
                            table_hbm.at[sidx_v.at[i + 1]], bufs[1], sem_g)
                    else:
                        @pl.when(j < ncw // 2 - 1)
                        def _():
                            pltpu.async_copy(
                                idx_hbm.at[1, base_row + i + 1], dbufs[0],
                                sem_i)
                            pltpu.async_copy(
                                table_hbm.at[sidx_v.at[i + 1]], bufs[0],
                                sem_g)
                return carry

            lax.fori_loop(0, ncw // 2, group, 0)
            pltpu.make_async_copy(bufs[0], y_sh.at[dbufs[0]], sem_s).wait()

        @pl.when(c == 0)
        def _():
            emit(s * ca, ca)

        @pl.when(c == 1)
        def _():
            emit(NS * ca + s * cb, cb)

        plsc.subcore_barrier()
        nfull = SLAB // CHUNK
        for k in range(nfull):
            pltpu.sync_copy(y_sh.at[pl.ds(s * SLAB + k * CHUNK, CHUNK)], b0)
            pltpu.sync_copy(b0, out_hbm.at[c, pl.ds(s * SLAB + k * CHUNK,
                                                    CHUNK)])
        if rem:
            pltpu.sync_copy(
                y_sh.at[pl.ds(s * SLAB + nfull * CHUNK, rem)],
                b0.at[pl.ds(0, rem)])
            pltpu.sync_copy(
                b0.at[pl.ds(0, rem)],
                out_hbm.at[c, pl.ds(s * SLAB + nfull * CHUNK, rem)])

    return functools.partial(
        pl.kernel,
        out_type=jax.ShapeDtypeStruct((NC, NPAD, D), jnp.float32),
        mesh=_mesh,
        compiler_params=_sc_params,
        scratch_types=[
            pltpu.VMEM((ca, CHUNK), jnp.int32),
            pltpu.VMEM((CHUNK,), jnp.int32),
            pltpu.VMEM((CHUNK,), jnp.int32),
            pltpu.VMEM((CHUNK, D), jnp.float32),
            pltpu.VMEM((CHUNK, D), jnp.float32),
            pltpu.VMEM_SHARED((NPAD, D), jnp.float32),
            pltpu.SemaphoreType.DMA,
            pltpu.SemaphoreType.DMA,
            pltpu.SemaphoreType.DMA,
        ],
    )(body)


_scatter128 = _make_scatter(D_HID, CA128, CB128)
_scatter32 = _make_scatter(N_CLASSES, CA32, CB32)


def _tc1_body(x_ref, w1_ref, dis_ref, xs1_ref):
    xw = jnp.dot(x_ref[...], w1_ref[...], preferred_element_type=jnp.float32)
    xs1_ref[...] = xw * dis_ref[...]


def _tc2_body(yp_ref, xs1_ref, dis_ref, b1_ref, w2_ref, xs2_ref):
    acc = yp_ref[0, :N_NODES, :] + yp_ref[1, :N_NODES, :] + xs1_ref[...]
    h = jnp.maximum(acc * dis_ref[...] + b1_ref[...], 0.0)
    xs2_ref[...] = jnp.dot(h, w2_ref[...],
                           preferred_element_type=jnp.float32) * dis_ref[...]


def _tc3_body(yp_ref, xs2_ref, dis_ref, b2_ref, out_ref):
    logits = (yp_ref[0, :N_NODES, :] + yp_ref[1, :N_NODES, :]
              + xs2_ref[...]) * dis_ref[...] + b2_ref[...]
    m = jnp.max(logits, axis=1, keepdims=True)
    z = logits - m
    out_ref[...] = z - jnp.log(jnp.sum(jnp.exp(z), axis=1, keepdims=True))


def _tc_call(body, out_shape, *args):
    return pl.pallas_call(
        body,
        out_shape=jax.ShapeDtypeStruct(out_shape, jnp.float32),
    )(*args)


def kernel(x, edge_index, W1, b1, W2, b2):
    src = edge_index[0].astype(jnp.int32)
    dst = edge_index[1].astype(jnp.int32)
    pad = E_PAD - src.shape[0]
    src_p = jnp.concatenate([src, jnp.zeros((pad,), jnp.int32)])
    dst_p = jnp.concatenate([dst, jnp.full((pad,), NPAD - 1, jnp.int32)])
    idx_all = jnp.stack([src_p.reshape(TOTAL_CH, CHUNK),
                         dst_p.reshape(TOTAL_CH, CHUNK)])

    deg_parts = _deg_kernel(idx_all)
    deg = jnp.sum(deg_parts, axis=0)[:N_NODES] + 1.0
    dis = lax.rsqrt(deg)[:, None]

    xs1 = _tc_call(_tc1_body, (N_NODES, D_HID), x, W1, dis)
    y1 = _scatter128(xs1, idx_all)
    xs2 = _tc_call(_tc2_body, (N_NODES, N_CLASSES),
                   y1, xs1, dis, b1[None, :], W2)
    y2 = _scatter32(xs2, idx_all)
    return _tc_call(_tc3_body, (N_NODES, N_CLASSES),
                    y2, xs2, dis, b2[None, :])


# speedup vs baseline: 1.0056x; 1.0056x over previous
import functools

import jax
import jax.numpy as jnp
from jax import lax
from jax.experimental import pallas as pl
from jax.experimental.pallas import tpu as pltpu
from jax.experimental.pallas import tpu_sc as plsc

N_NODES = 10000
D_FEAT = 128
D_HID = 128
N_CLASSES = 32

NC = 2
NS = 16
NW = NC * NS
L = 16

NPAD = 10112
SLAB = NPAD // NS
CHUNK = 128
TOTAL_CH = 2560
E_PAD = TOTAL_CH * CHUNK

CA128, CB128 = 126, 34
CA32, CB32 = 106, 54
CAD, CBD = 102, 58

_mesh = plsc.VectorSubcoreMesh(core_axis_name="c", subcore_axis_name="s")
_sc_params = pltpu.CompilerParams(needs_layout_passes=False,
                                  use_tc_tiling_on_sc=False)


def _deg_body(idx_hbm, out_hbm, idx_v, deg_v):
    c = lax.axis_index("c")
    s = lax.axis_index("s")
    wid = s * NC + c
    zero16 = jnp.zeros((L,), jnp.float32)
    one16 = jnp.ones((L,), jnp.float32)

    def zero_body(i, carry):
        deg_v[pl.ds(i * L, L)] = zero16
        return carry

    lax.fori_loop(0, NPAD // L, zero_body, 0)

    def count(base_row, ncw):
        pltpu.sync_copy(idx_hbm.at[1, pl.ds(base_row, ncw)],
                        idx_v.at[pl.ds(0, ncw)])

        def count_chunk(i, carry):
            for j in range(CHUNK // L):
                idx = idx_v[i, pl.ds(j * L, L)]
                plsc.addupdate_scatter(deg_v, [idx], one16)
            return carry

        lax.fori_loop(0, ncw, count_chunk, 0)

    @pl.when(c == 0)
    def _():
        count(s * CAD, CAD)

    @pl.when(c == 1)
    def _():
        count(NS * CAD + s * CBD, CBD)

    pltpu.sync_copy(deg_v, out_hbm.at[wid])


_deg_kernel = functools.partial(
    pl.kernel,
    out_type=jax.ShapeDtypeStruct((NW, NPAD), jnp.float32),
    mesh=_mesh,
    compiler_params=_sc_params,
    scratch_types=[
        pltpu.VMEM((CAD, CHUNK), jnp.int32),
        pltpu.VMEM((NPAD,), jnp.float32),
    ],
)(_deg_body)


def _make_scatter(D, ca, cb):
    def body(table_hbm, idx_hbm, out_hbm, sidx_v, d0, d1, b0, b1, y_sh,
             sem_i, sem_g, sem_s):
        c = lax.axis_index("c")
        s = lax.axis_index("s")
        dbufs = (d0, d1)
        bufs = (b0, b1)
        zero16 = jnp.zeros((L,), jnp.float32)
        izero16 = jnp.zeros((L,), jnp.int32)

        def zero_row(i, carry):
            for j in range(D // L):
                b1[i, pl.ds(j * L, L)] = zero16
            return carry

        with jax.named_scope("ph_zero"):
            lax.fori_loop(0, CHUNK, zero_row, 0)
            for j in range(CHUNK // L):
                d1[pl.ds(j * L, L)] = izero16
            for j in range(SLAB // CHUNK):
                pltpu.sync_copy(b1,
                                y_sh.at[pl.ds(s * SLAB + j * CHUNK, CHUNK)])
            rem = SLAB - (SLAB // CHUNK) * CHUNK
            if rem:
                pltpu.sync_copy(
                    b1.at[pl.ds(0, rem)],
                    y_sh.at[pl.ds(s * SLAB + SLAB - rem, rem)])
            plsc.subcore_barrier()

        def emit(base_row, ncw):
            pltpu.sync_copy(idx_hbm.at[0, pl.ds(base_row, ncw)],
                            sidx_v.at[pl.ds(0, ncw)])
            pltpu.async_copy(b1, y_sh.at[d1], sem_s, add=True)
            pltpu.async_copy(idx_hbm.at[1, base_row], d0, sem_i)
            pltpu.async_copy(table_hbm.at[sidx_v.at[0]], b0, sem_g)

            def group(j, carry):
                for p in range(2):
                    i = 2 * j + p
                    pltpu.make_async_copy(
                        idx_hbm.at[1, 0], dbufs[p], sem_i).wait()
                    pltpu.make_async_copy(
                        table_hbm.at[sidx_v.at[0]], bufs[p], sem_g).wait()
                    pltpu.make_async_copy(
                        bufs[p], y_sh.at[dbufs[p]], sem_s).wait()
                    pltpu.async_copy(
                        bufs[p], y_sh.at[dbufs[p]], sem_s, add=True)

                    if p == 0:
                        pltpu.async_copy(
                            idx_hbm.at[1, base_row + i + 1], dbufs[1], sem_i)
                        pltpu.async_copy(
                            table_hbm.at[sidx_v.at[i + 1]], bufs[1], sem_g)
                    else:
                        @pl.when(j < ncw // 2 - 1)
                        def _():
                            pltpu.async_copy(
                                idx_hbm.at[1, base_row + i + 1], dbufs[0],
                                sem_i)
                            pltpu.async_copy(
                                table_hbm.at[sidx_v.at[i + 1]], bufs[0],
                                sem_g)
                return carry

            lax.fori_loop(0, ncw // 2, group, 0)
            pltpu.make_async_copy(bufs[0], y_sh.at[dbufs[0]], sem_s).wait()

        with jax.named_scope("ph_edges"):
            @pl.when(c == 0)
            def _():
                emit(s * ca, ca)

            @pl.when(c == 1)
            def _():
                emit(NS * ca + s * cb, cb)

            plsc.subcore_barrier()
        with jax.named_scope("ph_out"):
            nfull = SLAB // CHUNK
            for k in range(nfull):
                pltpu.sync_copy(y_sh.at[pl.ds(s * SLAB + k * CHUNK, CHUNK)],
                                b0)
                pltpu.sync_copy(b0, out_hbm.at[c, pl.ds(s * SLAB + k * CHUNK,
                                                        CHUNK)])
            if rem:
                pltpu.sync_copy(
                    y_sh.at[pl.ds(s * SLAB + nfull * CHUNK, rem)],
                    b0.at[pl.ds(0, rem)])
                pltpu.sync_copy(
                    b0.at[pl.ds(0, rem)],
                    out_hbm.at[c, pl.ds(s * SLAB + nfull * CHUNK, rem)])

    return functools.partial(
        pl.kernel,
        out_type=jax.ShapeDtypeStruct((NC, NPAD, D), jnp.float32),
        mesh=_mesh,
        compiler_params=_sc_params,
        scratch_types=[
            pltpu.VMEM((ca, CHUNK), jnp.int32),
            pltpu.VMEM((CHUNK,), jnp.int32),
            pltpu.VMEM((CHUNK,), jnp.int32),
            pltpu.VMEM((CHUNK, D), jnp.float32),
            pltpu.VMEM((CHUNK, D), jnp.float32),
            pltpu.VMEM_SHARED((NPAD, D), jnp.float32),
            pltpu.SemaphoreType.DMA,
            pltpu.SemaphoreType.DMA,
            pltpu.SemaphoreType.DMA,
        ],
    )(body)


_scatter128 = _make_scatter(D_HID, CA128, CB128)
_scatter32 = _make_scatter(N_CLASSES, CA32, CB32)


def _tc1_body(x_ref, w1_ref, dis_ref, xs1_ref):
    xw = jnp.dot(x_ref[...], w1_ref[...], preferred_element_type=jnp.float32)
    xs1_ref[...] = xw * dis_ref[...]


def _tc2_body(yp_ref, xs1_ref, dis_ref, b1_ref, w2_ref, xs2_ref):
    acc = yp_ref[0, :N_NODES, :] + yp_ref[1, :N_NODES, :] + xs1_ref[...]
    h = jnp.maximum(acc * dis_ref[...] + b1_ref[...], 0.0)
    xs2_ref[...] = jnp.dot(h, w2_ref[...],
                           preferred_element_type=jnp.float32) * dis_ref[...]


def _tc3_body(yp_ref, xs2_ref, dis_ref, b2_ref, out_ref):
    logits = (yp_ref[0, :N_NODES, :] + yp_ref[1, :N_NODES, :]
              + xs2_ref[...]) * dis_ref[...] + b2_ref[...]
    m = jnp.max(logits, axis=1, keepdims=True)
    z = logits - m
    out_ref[...] = z - jnp.log(jnp.sum(jnp.exp(z), axis=1, keepdims=True))


def _tc_call(body, out_shape, *args):
    return pl.pallas_call(
        body,
        out_shape=jax.ShapeDtypeStruct(out_shape, jnp.float32),
    )(*args)


def kernel(x, edge_index, W1, b1, W2, b2):
    src = edge_index[0].astype(jnp.int32)
    dst = edge_index[1].astype(jnp.int32)
    pad = E_PAD - src.shape[0]
    src_p = jnp.concatenate([src, jnp.zeros((pad,), jnp.int32)])
    dst_p = jnp.concatenate([dst, jnp.full((pad,), NPAD - 1, jnp.int32)])
    idx_all = jnp.stack([src_p.reshape(TOTAL_CH, CHUNK),
                         dst_p.reshape(TOTAL_CH, CHUNK)])

    deg_parts = _deg_kernel(idx_all)
    deg = jnp.sum(deg_parts, axis=0)[:N_NODES] + 1.0
    dis = lax.rsqrt(deg)[:, None]

    xs1 = _tc_call(_tc1_body, (N_NODES, D_HID), x, W1, dis)
    y1 = _scatter128(xs1, idx_all)
    xs2 = _tc_call(_tc2_body, (N_NODES, N_CLASSES),
                   y1, xs1, dis, b1[None, :], W2)
    y2 = _scatter32(xs2, idx_all)
    return _tc_call(_tc3_body, (N_NODES, N_CLASSES),
                    y2, xs2, dis, b2[None, :])


# speedup vs baseline: 2.5709x; 2.5566x over previous
import functools

import jax
import jax.numpy as jnp
from jax import lax
from jax.experimental import pallas as pl
from jax.experimental.pallas import tpu as pltpu
from jax.experimental.pallas import tpu_sc as plsc

N_NODES = 10000
D_FEAT = 128
D_HID = 128
N_CLASSES = 32

NC = 2
NS = 16
NW = NC * NS
L = 16

NPAD = 10112
SLAB = NPAD // NS
CHUNK = 128
TOTAL_CH = 2560
E_PAD = TOTAL_CH * CHUNK

CA128, CB128 = 80, 80
CA32, CB32 = 80, 80
CAD, CBD = 80, 80

_mesh = plsc.VectorSubcoreMesh(core_axis_name="c", subcore_axis_name="s")
_sc_params = pltpu.CompilerParams(needs_layout_passes=False,
                                  use_tc_tiling_on_sc=False)


def _deg_body(idx_hbm, out_hbm, idx_v, deg_v):
    c = lax.axis_index("c")
    s = lax.axis_index("s")
    wid = s * NC + c
    zero16 = jnp.zeros((L,), jnp.float32)
    one16 = jnp.ones((L,), jnp.float32)

    def zero_body(i, carry):
        deg_v[pl.ds(i * L, L)] = zero16
        return carry

    lax.fori_loop(0, NPAD // L, zero_body, 0)

    def count(base_row, ncw):
        pltpu.sync_copy(idx_hbm.at[1, pl.ds(base_row, ncw)],
                        idx_v.at[pl.ds(0, ncw)])

        def count_chunk(i, carry):
            for j in range(CHUNK // L):
                idx = idx_v[i, pl.ds(j * L, L)]
                plsc.addupdate_scatter(deg_v, [idx], one16)
            return carry

        lax.fori_loop(0, ncw, count_chunk, 0)

    @pl.when(c == 0)
    def _():
        count(s * CAD, CAD)

    @pl.when(c == 1)
    def _():
        count(NS * CAD + s * CBD, CBD)

    pltpu.sync_copy(deg_v, out_hbm.at[wid])


_deg_kernel = functools.partial(
    pl.kernel,
    out_type=jax.ShapeDtypeStruct((NW, NPAD), jnp.float32),
    mesh=_mesh,
    compiler_params=_sc_params,
    scratch_types=[
        pltpu.VMEM((CAD, CHUNK), jnp.int32),
        pltpu.VMEM((NPAD,), jnp.float32),
    ],
)(_deg_body)


def _make_scatter(D, ca, cb):
    def body(table_hbm, idx_hbm, out_hbm, sidx_v, d0, d1, b0, b1, y_sh,
             sem_i, sem_g, sem_s):
        c = lax.axis_index("c")
        s = lax.axis_index("s")
        dbufs = (d0, d1)
        bufs = (b0, b1)
        zero16 = jnp.zeros((L,), jnp.float32)
        izero16 = jnp.zeros((L,), jnp.int32)

        def zero_row(i, carry):
            for j in range(D // L):
                b1[i, pl.ds(j * L, L)] = zero16
            return carry

        with jax.named_scope("ph_zero"):
            lax.fori_loop(0, CHUNK, zero_row, 0)
            for j in range(CHUNK // L):
                d1[pl.ds(j * L, L)] = izero16
            for j in range(SLAB // CHUNK):
                pltpu.sync_copy(b1,
                                y_sh.at[pl.ds(s * SLAB + j * CHUNK, CHUNK)])
            rem = SLAB - (SLAB // CHUNK) * CHUNK
            if rem:
                pltpu.sync_copy(
                    b1.at[pl.ds(0, rem)],
                    y_sh.at[pl.ds(s * SLAB + SLAB - rem, rem)])
            plsc.subcore_barrier()

        def emit(base_row, ncw):
            pltpu.sync_copy(idx_hbm.at[0, pl.ds(base_row, ncw)],
                            sidx_v.at[pl.ds(0, ncw)])
            pltpu.async_copy(b1, y_sh.at[d1], sem_s, add=True)
            pltpu.async_copy(idx_hbm.at[1, base_row], d0, sem_i)
            pltpu.async_copy(table_hbm.at[sidx_v.at[0]], b0, sem_g)

            def group(j, carry):
                for p in range(2):
                    i = 2 * j + p
                    pltpu.make_async_copy(
                        idx_hbm.at[1, 0], dbufs[p], sem_i).wait()
                    pltpu.make_async_copy(
                        table_hbm.at[sidx_v.at[0]], bufs[p], sem_g).wait()
                    pltpu.make_async_copy(
                        bufs[p], y_sh.at[dbufs[p]], sem_s).wait()
                    pltpu.async_copy(
                        bufs[p], y_sh.at[dbufs[p]], sem_s, add=True)

                    if p == 0:
                        pltpu.async_copy(
                            idx_hbm.at[1, base_row + i + 1], dbufs[1], sem_i)
                        pltpu.async_copy(
                            table_hbm.at[sidx_v.at[i + 1]], bufs[1], sem_g)
                    else:
                        @pl.when(j < ncw // 2 - 1)
                        def _():
                            pltpu.async_copy(
                                idx_hbm.at[1, base_row + i + 1], dbufs[0],
                                sem_i)
                            pltpu.async_copy(
                                table_hbm.at[sidx_v.at[i + 1]], bufs[0],
                                sem_g)
                return carry

            lax.fori_loop(0, ncw // 2, group, 0)
            pltpu.make_async_copy(bufs[0], y_sh.at[dbufs[0]], sem_s).wait()

        with jax.named_scope("ph_edges"):
            @pl.when(c == 0)
            def _():
                emit(s * ca, ca)

            @pl.when(c == 1)
            def _():
                emit(NS * ca + s * cb, cb)

            plsc.subcore_barrier()
        with jax.named_scope("ph_out"):
            nfull = SLAB // CHUNK
            for k in range(nfull):
                pltpu.sync_copy(y_sh.at[pl.ds(s * SLAB + k * CHUNK, CHUNK)],
                                b0)
                pltpu.sync_copy(b0, out_hbm.at[c, pl.ds(s * SLAB + k * CHUNK,
                                                        CHUNK)])
            if rem:
                pltpu.sync_copy(
                    y_sh.at[pl.ds(s * SLAB + nfull * CHUNK, rem)],
                    b0.at[pl.ds(0, rem)])
                pltpu.sync_copy(
                    b0.at[pl.ds(0, rem)],
                    out_hbm.at[c, pl.ds(s * SLAB + nfull * CHUNK, rem)])

    return functools.partial(
        pl.kernel,
        out_type=jax.ShapeDtypeStruct((NC, NPAD, D), jnp.float32),
        mesh=_mesh,
        compiler_params=_sc_params,
        scratch_types=[
            pltpu.VMEM((ca, CHUNK), jnp.int32),
            pltpu.VMEM((CHUNK,), jnp.int32),
            pltpu.VMEM((CHUNK,), jnp.int32),
            pltpu.VMEM((CHUNK, D), jnp.float32),
            pltpu.VMEM((CHUNK, D), jnp.float32),
            pltpu.VMEM_SHARED((NPAD, D), jnp.float32),
            pltpu.SemaphoreType.DMA,
            pltpu.SemaphoreType.DMA,
            pltpu.SemaphoreType.DMA,
        ],
    )(body)


_scatter128 = _make_scatter(D_HID, CA128, CB128)
_scatter32 = _make_scatter(N_CLASSES, CA32, CB32)


def _tc1_body(x_ref, w1_ref, dis_ref, xs1_ref):
    xw = jnp.dot(x_ref[...], w1_ref[...], preferred_element_type=jnp.float32)
    xs1_ref[...] = xw * dis_ref[...]


def _tc2_body(yp_ref, xs1_ref, dis_ref, b1_ref, w2_ref, xs2_ref):
    acc = yp_ref[0, :N_NODES, :] + yp_ref[1, :N_NODES, :] + xs1_ref[...]
    h = jnp.maximum(acc * dis_ref[...] + b1_ref[...], 0.0)
    xs2_ref[...] = jnp.dot(h, w2_ref[...],
                           preferred_element_type=jnp.float32) * dis_ref[...]


def _tc3_body(yp_ref, xs2_ref, dis_ref, b2_ref, out_ref):
    logits = (yp_ref[0, :N_NODES, :] + yp_ref[1, :N_NODES, :]
              + xs2_ref[...]) * dis_ref[...] + b2_ref[...]
    m = jnp.max(logits, axis=1, keepdims=True)
    z = logits - m
    out_ref[...] = z - jnp.log(jnp.sum(jnp.exp(z), axis=1, keepdims=True))


def _tc_call(body, out_shape, *args):
    return pl.pallas_call(
        body,
        out_shape=jax.ShapeDtypeStruct(out_shape, jnp.float32),
    )(*args)


def kernel(x, edge_index, W1, b1, W2, b2):
    src = edge_index[0].astype(jnp.int32)
    dst = edge_index[1].astype(jnp.int32)
    pad = E_PAD - src.shape[0]
    pad_ar = jnp.arange(pad, dtype=jnp.int32)
    src_p = jnp.concatenate([src, (pad_ar * 79) % N_NODES])
    dst_p = jnp.concatenate([dst, N_NODES + pad_ar % (NPAD - N_NODES)])
    idx_all = jnp.stack([src_p.reshape(TOTAL_CH, CHUNK),
                         dst_p.reshape(TOTAL_CH, CHUNK)])

    deg_parts = _deg_kernel(idx_all)
    deg = jnp.sum(deg_parts, axis=0)[:N_NODES] + 1.0
    dis = lax.rsqrt(deg)[:, None]

    xs1 = _tc_call(_tc1_body, (N_NODES, D_HID), x, W1, dis)
    y1 = _scatter128(xs1, idx_all)
    xs2 = _tc_call(_tc2_body, (N_NODES, N_CLASSES),
                   y1, xs1, dis, b1[None, :], W2)
    y2 = _scatter32(xs2, idx_all)
    return _tc_call(_tc3_body, (N_NODES, N_CLASSES),
                    y2, xs2, dis, b2[None, :])


# speedup vs baseline: 2.6118x; 1.0159x over previous
import functools

import jax
import jax.numpy as jnp
from jax import lax
from jax.experimental import pallas as pl
from jax.experimental.pallas import tpu as pltpu
from jax.experimental.pallas import tpu_sc as plsc

N_NODES = 10000
D_FEAT = 128
D_HID = 128
N_CLASSES = 32

NC = 2
NS = 16
NW = NC * NS
L = 16

NPAD = 10112
SLAB = NPAD // NS
CHUNK = 128
TOTAL_CH = 2560
E_PAD = TOTAL_CH * CHUNK

CA128, CB128 = 80, 80
CA32, CB32 = 80, 80
CAD, CBD = 80, 80

_mesh = plsc.VectorSubcoreMesh(core_axis_name="c", subcore_axis_name="s")
_sc_params = pltpu.CompilerParams(needs_layout_passes=False,
                                  use_tc_tiling_on_sc=False)


def _deg_body(didx_hbm, out_hbm, idx_v, deg_v):
    c = lax.axis_index("c")
    s = lax.axis_index("s")
    wid = s * NC + c
    zero16 = jnp.zeros((L,), jnp.float32)
    one16 = jnp.ones((L,), jnp.float32)

    def zero_body(i, carry):
        deg_v[pl.ds(i * L, L)] = zero16
        return carry

    lax.fori_loop(0, NPAD // L, zero_body, 0)

    def count(base_row, ncw):
        pltpu.sync_copy(didx_hbm.at[pl.ds(base_row, ncw)],
                        idx_v.at[pl.ds(0, ncw)])

        def count_chunk(i, carry):
            for j in range(CHUNK // L):
                idx = idx_v[i, pl.ds(j * L, L)]
                plsc.addupdate_scatter(deg_v, [idx], one16)
            return carry

        lax.fori_loop(0, ncw, count_chunk, 0)

    @pl.when(c == 0)
    def _():
        count(s * CAD, CAD)

    @pl.when(c == 1)
    def _():
        count(NS * CAD + s * CBD, CBD)

    pltpu.sync_copy(deg_v, out_hbm.at[wid])


_deg_kernel = functools.partial(
    pl.kernel,
    out_type=jax.ShapeDtypeStruct((NW, NPAD), jnp.float32),
    mesh=_mesh,
    compiler_params=_sc_params,
    scratch_types=[
        pltpu.VMEM((CAD, CHUNK), jnp.int32),
        pltpu.VMEM((NPAD,), jnp.float32),
    ],
)(_deg_body)


def _make_scatter(D, ca, cb):
    def body(table_hbm, sidx_hbm, didx_hbm, out_hbm, sidx_v, d0, d1, b0, b1, y_sh,
             sem_i, sem_g, sem_s):
        c = lax.axis_index("c")
        s = lax.axis_index("s")
        dbufs = (d0, d1)
        bufs = (b0, b1)
        zero16 = jnp.zeros((L,), jnp.float32)
        izero16 = jnp.zeros((L,), jnp.int32)

        def zero_row(i, carry):
            for j in range(D // L):
                b1[i, pl.ds(j * L, L)] = zero16
            return carry

        with jax.named_scope("ph_zero"):
            lax.fori_loop(0, CHUNK, zero_row, 0)
            for j in range(CHUNK // L):
                d1[pl.ds(j * L, L)] = izero16
            for j in range(SLAB // CHUNK):
                pltpu.sync_copy(b1,
                                y_sh.at[pl.ds(s * SLAB + j * CHUNK, CHUNK)])
            rem = SLAB - (SLAB // CHUNK) * CHUNK
            if rem:
                pltpu.sync_copy(
                    b1.at[pl.ds(0, rem)],
                    y_sh.at[pl.ds(s * SLAB + SLAB - rem, rem)])
            plsc.subcore_barrier()

        def emit(base_row, ncw):
            pltpu.sync_copy(sidx_hbm.at[pl.ds(base_row, ncw)],
                            sidx_v.at[pl.ds(0, ncw)])
            pltpu.async_copy(b1, y_sh.at[d1], sem_s, add=True)
            pltpu.async_copy(didx_hbm.at[base_row], d0, sem_i)
            pltpu.async_copy(table_hbm.at[sidx_v.at[0]], b0, sem_g)

            def group(j, carry):
                for p in range(2):
                    i = 2 * j + p
                    pltpu.make_async_copy(
                        didx_hbm.at[0], dbufs[p], sem_i).wait()
                    pltpu.make_async_copy(
                        table_hbm.at[sidx_v.at[0]], bufs[p], sem_g).wait()
                    pltpu.make_async_copy(
                        bufs[p], y_sh.at[dbufs[p]], sem_s).wait()
                    pltpu.async_copy(
                        bufs[p], y_sh.at[dbufs[p]], sem_s, add=True)

                    if p == 0:
                        pltpu.async_copy(
                            didx_hbm.at[base_row + i + 1], dbufs[1], sem_i)
                        pltpu.async_copy(
                            table_hbm.at[sidx_v.at[i + 1]], bufs[1], sem_g)
                    else:
                        @pl.when(j < ncw // 2 - 1)
                        def _():
                            pltpu.async_copy(
                                didx_hbm.at[base_row + i + 1], dbufs[0],
                                sem_i)
                            pltpu.async_copy(
                                table_hbm.at[sidx_v.at[i + 1]], bufs[0],
                                sem_g)
                return carry

            lax.fori_loop(0, ncw // 2, group, 0)
            pltpu.make_async_copy(bufs[0], y_sh.at[dbufs[0]], sem_s).wait()

        with jax.named_scope("ph_edges"):
            @pl.when(c == 0)
            def _():
                emit(s * ca, ca)

            @pl.when(c == 1)
            def _():
                emit(NS * ca + s * cb, cb)

            plsc.subcore_barrier()
        with jax.named_scope("ph_out"):
            nfull = SLAB // CHUNK
            for k in range(nfull):
                pltpu.sync_copy(y_sh.at[pl.ds(s * SLAB + k * CHUNK, CHUNK)],
                                b0)
                pltpu.sync_copy(b0, out_hbm.at[c, pl.ds(s * SLAB + k * CHUNK,
                                                        CHUNK)])
            if rem:
                pltpu.sync_copy(
                    y_sh.at[pl.ds(s * SLAB + nfull * CHUNK, rem)],
                    b0.at[pl.ds(0, rem)])
                pltpu.sync_copy(
                    b0.at[pl.ds(0, rem)],
                    out_hbm.at[c, pl.ds(s * SLAB + nfull * CHUNK, rem)])

    return functools.partial(
        pl.kernel,
        out_type=jax.ShapeDtypeStruct((NC, NPAD, D), jnp.float32),
        mesh=_mesh,
        compiler_params=_sc_params,
        scratch_types=[
            pltpu.VMEM((ca, CHUNK), jnp.int32),
            pltpu.VMEM((CHUNK,), jnp.int32),
            pltpu.VMEM((CHUNK,), jnp.int32),
            pltpu.VMEM((CHUNK, D), jnp.float32),
            pltpu.VMEM((CHUNK, D), jnp.float32),
            pltpu.VMEM_SHARED((NPAD, D), jnp.float32),
            pltpu.SemaphoreType.DMA,
            pltpu.SemaphoreType.DMA,
            pltpu.SemaphoreType.DMA,
        ],
    )(body)


_scatter128 = _make_scatter(D_HID, CA128, CB128)
_scatter32 = _make_scatter(N_CLASSES, CA32, CB32)


def _tc1_body(parts_ref, x_ref, w1_ref, xs1_ref, dis_ref):
    deg = jnp.sum(parts_ref[...], axis=0)[:N_NODES] + 1.0
    dis = lax.rsqrt(deg)[:, None]
    dis_ref[...] = dis
    xw = jnp.dot(x_ref[...], w1_ref[...], preferred_element_type=jnp.float32)
    xs1_ref[...] = xw * dis


def _tc2_body(yp_ref, xs1_ref, dis_ref, b1_ref, w2_ref, xs2_ref):
    acc = yp_ref[0, :N_NODES, :] + yp_ref[1, :N_NODES, :] + xs1_ref[...]
    h = jnp.maximum(acc * dis_ref[...] + b1_ref[...], 0.0)
    xs2_ref[...] = jnp.dot(h, w2_ref[...],
                           preferred_element_type=jnp.float32) * dis_ref[...]


def _tc3_body(yp_ref, xs2_ref, dis_ref, b2_ref, out_ref):
    logits = (yp_ref[0, :N_NODES, :] + yp_ref[1, :N_NODES, :]
              + xs2_ref[...]) * dis_ref[...] + b2_ref[...]
    m = jnp.max(logits, axis=1, keepdims=True)
    z = logits - m
    out_ref[...] = z - jnp.log(jnp.sum(jnp.exp(z), axis=1, keepdims=True))


def _tc_call(body, out_shape, *args):
    if isinstance(out_shape, list):
        shapes = [jax.ShapeDtypeStruct(s, jnp.float32) for s in out_shape]
    else:
        shapes = jax.ShapeDtypeStruct(out_shape, jnp.float32)
    return pl.pallas_call(body, out_shape=shapes)(*args)


def kernel(x, edge_index, W1, b1, W2, b2):
    src = edge_index[0].astype(jnp.int32)
    dst = edge_index[1].astype(jnp.int32)
    pad = E_PAD - src.shape[0]
    pad_ar = jnp.arange(pad, dtype=jnp.int32)
    src_c = jnp.concatenate(
        [src, (pad_ar * 79) % N_NODES]).reshape(TOTAL_CH, CHUNK)
    dst_c = jnp.concatenate(
        [dst, N_NODES + pad_ar % (NPAD - N_NODES)]).reshape(TOTAL_CH, CHUNK)

    deg_parts = _deg_kernel(dst_c)
    xs1, dis = _tc_call(_tc1_body,
                        [(N_NODES, D_HID), (N_NODES, 1)],
                        deg_parts, x, W1)
    y1 = _scatter128(xs1, src_c, dst_c)
    xs2 = _tc_call(_tc2_body, (N_NODES, N_CLASSES),
                   y1, xs1, dis, b1[None, :], W2)
    y2 = _scatter32(xs2, src_c, dst_c)
    return _tc_call(_tc3_body, (N_NODES, N_CLASSES),
                    y2, xs2, dis, b2[None, :])
